# baseline (speedup 1.0000x reference)
import jax
import jax.numpy as jnp
import numpy as np
from jax.experimental import pallas as pl
from jax.experimental.pallas import tpu as pltpu

_B, _D, _S, _K = 4096, 256, 64, 64
_BB = 64
_R = _BB * _S


def _dot(x, w):
    return jax.lax.dot_general(
        x, w, (((1,), (0,)), ((), ())),
        precision=jax.lax.Precision.HIGHEST,
        preferred_element_type=jnp.float32)


def _dotT(x, w):
    return jax.lax.dot_general(
        x, w, (((1,), (1,)), ((), ())),
        precision=jax.lax.Precision.HIGHEST,
        preferred_element_type=jnp.float32)


def _body(wp_ref, hidden_ref, keys_ref, values_ref, wq_ref, bq_ref, wk_ref,
          bk_ref, wv_ref, bv_ref, wo_ref, bo_ref,
          read_ref, nk_ref, nv_ref, nptr_ref):
    h = hidden_ref[...]
    kb = keys_ref[...]
    vb = values_ref[...]
    wpi = wp_ref[...]

    seg_of_row = jax.lax.broadcasted_iota(jnp.int32, (_R, _BB), 0) // _S
    col = jax.lax.broadcasted_iota(jnp.int32, (_R, _BB), 1)
    E = (seg_of_row == col).astype(jnp.float32)
    seg_of_rowT = jax.lax.broadcasted_iota(jnp.int32, (_BB, _R), 1) // _S
    rowT = jax.lax.broadcasted_iota(jnp.int32, (_BB, _R), 0)
    ET = (seg_of_rowT == rowT).astype(jnp.float32)

    q = _dotT(h, wq_ref[...]) + bq_ref[...]
    q_e = _dot(E, q)
    logits = jnp.sum(kb * q_e, axis=1, keepdims=True)
    e = jnp.exp(logits * np.float32(1.0 / np.sqrt(_K)))
    seg_sum = _dot(ET, e)
    den = _dot(E, seg_sum)
    p = e / den
    readv = _dot(ET, p * vb)
    read_ref[...] = _dotT(readv, wo_ref[...]) + bo_ref[...]

    nk = _dotT(h, wk_ref[...]) + bk_ref[...]
    nv = _dotT(h, wv_ref[...]) + bv_ref[...]
    wp_e = _dot(E, wpi.astype(jnp.float32))
    s_idx = (jax.lax.broadcasted_iota(jnp.int32, (_R, 1), 0) % _S
             ).astype(jnp.float32)
    hit = s_idx == wp_e
    nk_ref[...] = jnp.where(hit, _dot(E, nk), kb)
    nv_ref[...] = jnp.where(hit, _dot(E, nv), vb)
    nptr_ref[...] = (wpi + 1) % _S


def kernel(hidden, keys, values, write_ptr, Wq, bq, Wk, bk, Wv, bv, Wo, bo):
    nb = _B // _BB
    wp2 = write_ptr.astype(jnp.int32).reshape(_B, 1)
    keys2 = keys.reshape(_B * _S, _K)
    values2 = values.reshape(_B * _S, _D)
    full = lambda shp: pl.BlockSpec(shp, lambda i: (0,) * len(shp))
    out = pl.pallas_call(
        _body,
        grid=(nb,),
        in_specs=[
            pl.BlockSpec((_BB, 1), lambda i: (i, 0)),
            pl.BlockSpec((_BB, _D), lambda i: (i, 0)),
            pl.BlockSpec((_R, _K), lambda i: (i, 0)),
            pl.BlockSpec((_R, _D), lambda i: (i, 0)),
            full((_K, _D)), full((1, _K)),
            full((_K, _D)), full((1, _K)),
            full((_D, _D)), full((1, _D)),
            full((_D, _D)), full((1, _D)),
        ],
        out_specs=[
            pl.BlockSpec((_BB, _D), lambda i: (i, 0)),
            pl.BlockSpec((_R, _K), lambda i: (i, 0)),
            pl.BlockSpec((_R, _D), lambda i: (i, 0)),
            pl.BlockSpec((_BB, 1), lambda i: (i, 0)),
        ],
        out_shape=[
            jax.ShapeDtypeStruct((_B, _D), jnp.float32),
            jax.ShapeDtypeStruct((_B * _S, _K), jnp.float32),
            jax.ShapeDtypeStruct((_B * _S, _D), jnp.float32),
            jax.ShapeDtypeStruct((_B, 1), jnp.int32),
        ],
        compiler_params=pltpu.CompilerParams(
            dimension_semantics=("arbitrary",)),
    )(wp2, hidden, keys2, values2, Wq, bq.reshape(1, _K), Wk,
      bk.reshape(1, _K), Wv, bv.reshape(1, _D), Wo, bo.reshape(1, _D))
    read, nk2, nv2, nptr = out
    return (read, nk2.reshape(_B, _S, _K), nv2.reshape(_B, _S, _D),
            nptr.reshape(_B))


# speedup vs baseline: 3.6454x; 3.6454x over previous
import jax
import jax.numpy as jnp
import numpy as np
from jax.experimental import pallas as pl
from jax.experimental.pallas import tpu as pltpu

_B, _D, _S, _K = 4096, 256, 64, 64
_BB = 64
_R = _BB * _S


def _dot(x, w, prec=jax.lax.Precision.DEFAULT):
    return jax.lax.dot_general(
        x, w, (((1,), (0,)), ((), ())),
        precision=prec, preferred_element_type=jnp.float32)


def _dotT(x, w):
    return jax.lax.dot_general(
        x, w, (((1,), (1,)), ((), ())),
        precision=jax.lax.Precision.HIGHEST,
        preferred_element_type=jnp.float32)


def _body(wps_ref, wp_ref, hidden_ref, keys_ref, values_ref, wq_ref, bq_ref,
          wk_ref, bk_ref, wv_ref, bv_ref, wo_ref, bo_ref,
          read_ref, nk_ref, nv_ref, nptr_ref):
    h = hidden_ref[...]
    kb = keys_ref[...]
    vb = values_ref[...]
    wpi = wp_ref[...]

    seg_of_row = jax.lax.broadcasted_iota(jnp.int32, (_R, _BB), 0) // _S
    col = jax.lax.broadcasted_iota(jnp.int32, (_R, _BB), 1)
    E = (seg_of_row == col).astype(jnp.float32)
    seg_of_rowT = jax.lax.broadcasted_iota(jnp.int32, (_BB, _R), 1) // _S
    rowT = jax.lax.broadcasted_iota(jnp.int32, (_BB, _R), 0)
    ET = (seg_of_rowT == rowT).astype(jnp.float32)

    q = _dotT(h, wq_ref[...]) + bq_ref[...]
    q_e = _dot(E, q)
    logits = jnp.sum(kb * q_e, axis=1, keepdims=True)
    e = jnp.exp(logits * np.float32(1.0 / np.sqrt(_K)))
    seg_sum = _dot(ET, e)
    den = _dot(E, seg_sum)
    p = e / den
    readv = _dot(ET, p * vb, jax.lax.Precision.DEFAULT)
    read_ref[...] = _dotT(readv, wo_ref[...]) + bo_ref[...]

    nk = _dotT(h, wk_ref[...]) + bk_ref[...]
    nv = _dotT(h, wv_ref[...]) + bv_ref[...]
    nk_ref[...] = kb
    nv_ref[...] = vb
    for i in range(_BB):
        base = i * _S + wps_ref[i, 0]
        nk_ref[pl.ds(base, 1), :] = nk[i:i + 1, :]
        nv_ref[pl.ds(base, 1), :] = nv[i:i + 1, :]
    nptr_ref[...] = (wpi + 1) % _S


def kernel(hidden, keys, values, write_ptr, Wq, bq, Wk, bk, Wv, bv, Wo, bo):
    nb = _B // _BB
    wp2 = write_ptr.astype(jnp.int32).reshape(_B, 1)
    keys2 = keys.reshape(_B * _S, _K)
    values2 = values.reshape(_B * _S, _D)
    full = lambda shp: pl.BlockSpec(shp, lambda i: (0,) * len(shp))
    out = pl.pallas_call(
        _body,
        grid=(nb,),
        in_specs=[
            pl.BlockSpec((_BB, 1), lambda i: (i, 0),
                         memory_space=pltpu.SMEM),
            pl.BlockSpec((_BB, 1), lambda i: (i, 0)),
            pl.BlockSpec((_BB, _D), lambda i: (i, 0)),
            pl.BlockSpec((_R, _K), lambda i: (i, 0)),
            pl.BlockSpec((_R, _D), lambda i: (i, 0)),
            full((_K, _D)), full((1, _K)),
            full((_K, _D)), full((1, _K)),
            full((_D, _D)), full((1, _D)),
            full((_D, _D)), full((1, _D)),
        ],
        out_specs=[
            pl.BlockSpec((_BB, _D), lambda i: (i, 0)),
            pl.BlockSpec((_R, _K), lambda i: (i, 0)),
            pl.BlockSpec((_R, _D), lambda i: (i, 0)),
            pl.BlockSpec((_BB, 1), lambda i: (i, 0)),
        ],
        out_shape=[
            jax.ShapeDtypeStruct((_B, _D), jnp.float32),
            jax.ShapeDtypeStruct((_B * _S, _K), jnp.float32),
            jax.ShapeDtypeStruct((_B * _S, _D), jnp.float32),
            jax.ShapeDtypeStruct((_B, 1), jnp.int32),
        ],
        compiler_params=pltpu.CompilerParams(
            dimension_semantics=("arbitrary",)),
    )(wp2, wp2, hidden, keys2, values2, Wq, bq.reshape(1, _K), Wk,
      bk.reshape(1, _K), Wv, bv.reshape(1, _D), Wo, bo.reshape(1, _D))
    read, nk2, nv2, nptr = out
    return (read, nk2.reshape(_B, _S, _K), nv2.reshape(_B, _S, _D),
            nptr.reshape(_B))
